# baseline (speedup 1.0000x reference)
import jax
import jax.numpy as jnp
from jax import lax
from jax.experimental import pallas as pl
from jax.experimental.pallas import tpu as pltpu
from jax.experimental.pallas import tpu_sc as plsc

_TS = 512
_B = 64
_NC, _NS, _L = 2, 16, 16
_ROWS = _TS // _NS
_CHUNK = _ROWS * _TS
_SLICE = _TS * _TS
_BPS = _B // _NC // _NS


def _sc_body(out_hbm, chunk_v, shared, sem, sem2):
    cid = lax.axis_index("c")
    sid = lax.axis_index("s")
    row0 = sid * _ROWS

    def row_body(r, _):
        d = row0 + r
        limit = _TS - d
        m1 = (d - _TS // 4) >> 31
        m2 = (d - _TS // 2) >> 31

        def col_body(c, _):
            s = lax.iota(jnp.int32, _L) + c * _L
            lt = (s - limit) >> 31
            even = ((s & 1) - 1) >> 31
            mod4 = ((s & 3) - 1) >> 31
            stride = m1 | (m2 & even) | mod4
            bits = lt & stride & jnp.int32(0x3F800000)
            chunk_v[pl.ds(r * _TS + c * _L, _L)] = lax.bitcast_convert_type(
                bits, jnp.float32
            )
            return 0

        return lax.fori_loop(0, _TS // _L, col_body, 0)

    lax.fori_loop(0, _ROWS, row_body, 0)

    pltpu.sync_copy(chunk_v, shared.at[pl.ds(row0 * _TS, _CHUNK)])
    plsc.subcore_barrier()

    b0 = (cid * _NS + sid) * _BPS

    def fire(i, _):
        pltpu.async_copy(shared, out_hbm.at[pl.ds((b0 + i) * _SLICE, _SLICE)], sem)
        return 0

    lax.fori_loop(0, _BPS, fire, 0)

    def drain(i, _):
        pltpu.make_async_copy(
            shared, out_hbm.at[pl.ds((b0 + i) * _SLICE, _SLICE)], sem
        ).wait()
        return 0

    lax.fori_loop(0, _BPS, drain, 0)


def kernel(start, end, actionness):
    f = pl.kernel(
        _sc_body,
        out_type=jax.ShapeDtypeStruct((_B * _TS * _TS,), jnp.float32),
        mesh=plsc.VectorSubcoreMesh(core_axis_name="c", subcore_axis_name="s"),
        scratch_types=[
            pltpu.VMEM((_CHUNK,), jnp.float32),
            pltpu.VMEM_SHARED((_SLICE,), jnp.float32),
            pltpu.SemaphoreType.DMA,
            pltpu.SemaphoreType.DMA,
        ],
    )
    return f().reshape(_B, _TS, _TS)


# speedup vs baseline: 6.0868x; 6.0868x over previous
import jax
import jax.numpy as jnp
from jax.experimental import pallas as pl

_TSCALE = 512


def _prop_mask_kernel(o_ref):
    ts = _TSCALE
    d = jax.lax.broadcasted_iota(jnp.int32, (ts, ts), 0)
    s = jax.lax.broadcasted_iota(jnp.int32, (ts, ts), 1)
    cond = ((d + s) < ts) & (
        (d < ts // 4)
        | ((d < ts // 2) & ((s & 1) == 0))
        | ((s & 3) == 0)
    )
    block = jnp.where(cond, 1.0, 0.0).astype(jnp.float32)
    o_ref[...] = jnp.broadcast_to(block[None], o_ref.shape)


def kernel(start, end, actionness):
    B = start.shape[0]
    ts = _TSCALE
    bb = 4
    return pl.pallas_call(
        _prop_mask_kernel,
        grid=(B // bb,),
        out_specs=pl.BlockSpec((bb, ts, ts), lambda i: (i, 0, 0)),
        out_shape=jax.ShapeDtypeStruct((B, ts, ts), jnp.float32),
    )()
